# baseline (speedup 1.0000x reference)
import functools

import jax
import jax.numpy as jnp
from jax import lax
from jax.experimental import pallas as pl
from jax.experimental.pallas import tpu as pltpu
from jax.experimental.pallas import tpu_sc as plsc

_CORPUS = 1_000_000
_H = 64
_CTX = 20
_NEG = 5
_T = _CTX + _NEG
_B = 16384

_NC = 2
_NS = 16
_NW = _NC * _NS
_NPW = _B // _NW

_CB = 16
_NCHUNK = _NPW // _CB
_G = 100
_GPC = _CB * _T // _G
_L = 16
_TP = 32


def _sc_body(idx0_hbm, idx1_hbm, w0_hbm, w1_hbm, out_hbm,
             idxe_v, idxt_v, emb_v, tgt_v, out_v, sem):
  wid = lax.axis_index("s") * _NC + lax.axis_index("c")

  def chunk_body(k, _):
    pltpu.sync_copy(idx0_hbm.at[wid * _NCHUNK + k], idxe_v)
    for q in range(_GPC):
      pltpu.sync_copy(
          idx1_hbm.at[(wid * _NCHUNK + k) * _GPC + q], idxt_v[q])
    cps = [pltpu.async_copy(w0_hbm.at[idxe_v], emb_v, sem)]
    for q in range(_GPC):
      cps.append(pltpu.async_copy(
          w1_hbm.at[idxt_v[q]], tgt_v.at[pl.ds(q * _G, _G)], sem))
    for cp in cps:
      cp.wait()

    lanes = lax.iota(jnp.int32, _L)

    def elem_body(i, _):
      e = [emb_v[i, pl.ds(j * _L, _L)] for j in range(_H // _L)]
      for g in range(_TP // _L):
        acc = jnp.zeros((_L,), jnp.float32)
        for cc in range(_L):
          c = g * _L + cc
          if c >= _T:
            break
          r = i * _T + c
          p = e[0] * tgt_v[r, pl.ds(0, _L)]
          for j in range(1, _H // _L):
            p = p + e[j] * tgt_v[r, pl.ds(j * _L, _L)]
          acc = jnp.where(lanes == cc, jnp.sum(p), acc)
        out_v[i, pl.ds(g * _L, _L)] = acc
      return _

    lax.fori_loop(0, _CB, elem_body, None)
    pltpu.sync_copy(out_v, out_hbm.at[pl.ds(wid * _NPW + k * _CB, _CB)])
    return _

  lax.fori_loop(0, _NCHUNK, chunk_body, None)


@jax.jit
def kernel(x, W0, W1):
  assert x.shape == (_B, 1 + _CTX)
  neg = jax.random.randint(jax.random.key(42), (_B, _NEG), 0, _CORPUS)
  idx0 = x[:, 0].reshape(_B // _CB, _CB)
  idx1 = jnp.concatenate(
      [x[:, 1:], neg.astype(jnp.int32)], axis=1).reshape(_B * _T // _G, _G)

  run = pl.kernel(
      _sc_body,
      out_type=jax.ShapeDtypeStruct((_B, _TP), jnp.float32),
      mesh=plsc.VectorSubcoreMesh(core_axis_name="c", subcore_axis_name="s",
                                  num_cores=_NC, num_subcores=_NS),
      compiler_params=pltpu.CompilerParams(needs_layout_passes=False,
                                           use_tc_tiling_on_sc=False),
      scratch_types=[
          pltpu.VMEM((_CB,), jnp.int32),
          [pltpu.VMEM((_G,), jnp.int32) for _ in range(_GPC)],
          pltpu.VMEM((_CB, _H), jnp.float32),
          pltpu.VMEM((_CB * _T, _H), jnp.float32),
          pltpu.VMEM((_CB, _TP), jnp.float32),
          pltpu.SemaphoreType.DMA,
      ],
  )
  return run(idx0, idx1, W0, W1)[:, :_T]
